# Initial kernel scaffold; baseline (speedup 1.0000x reference)
import jax
import jax.numpy as jnp
from jax.experimental import pallas as pl


def kernel(x, edge_index, batch_nodes, labels, W1, b1, W2, b2, Wc, bc, Wl, bl):
    raise NotImplementedError("write your pallas kernel here")


# speedup vs baseline: 7.9815x; 7.9815x over previous
import functools

import jax
import jax.numpy as jnp
from jax import lax
from jax.experimental import pallas as pl
from jax.experimental.pallas import tpu as pltpu
from jax.experimental.pallas import tpu_sc as plsc

NNODES = 10000
NP = 10240
H = 128
EDGES = 320000
BATCH = 1024
NC = 2
NS = 16
NW = NC * NS
K = 128
CH = 79
EPW = CH * K
EP = NW * EPW
RPS = NP // NS
RB = 1024
NB = NP // RB
BPW = BATCH // NW

_MM = dict(preferred_element_type=jnp.float32, precision=lax.Precision.HIGHEST)


def _segsum_kernel(width):
    def body(table, src3, dst3, out, srcv, dstv, rows, acc):
        c = lax.axis_index("c")
        s = lax.axis_index("s")
        wid = c * NS + s
        base = s * RPS
        pltpu.sync_copy(table.at[pl.ds(base, RPS)], acc.at[pl.ds(base, RPS)])
        pltpu.sync_copy(src3.at[wid], srcv)
        pltpu.sync_copy(dst3.at[wid], dstv)
        plsc.subcore_barrier()

        @pl.loop(0, CH)
        def _(j):
            pltpu.sync_copy(table.at[srcv.at[j]], rows)
            pltpu.sync_copy(rows, acc.at[dstv.at[j]], add=True)

        plsc.subcore_barrier()
        pltpu.sync_copy(acc.at[pl.ds(base, RPS)], out.at[c, pl.ds(base, RPS)])

    return pl.kernel(
        body,
        out_type=jax.ShapeDtypeStruct((NC, NP, width), jnp.float32),
        mesh=plsc.VectorSubcoreMesh(core_axis_name="c", subcore_axis_name="s"),
        scratch_types=[
            pltpu.VMEM((CH, K), jnp.int32),
            pltpu.VMEM((CH, K), jnp.int32),
            pltpu.VMEM((K, width), jnp.float32),
            pltpu.VMEM_SHARED((NP, width), jnp.float32),
        ],
    )


_segsum_feat = _segsum_kernel(H)


def _deg_body(zeros, ones128, dst3, out, dstv, onesv, acc):
    c = lax.axis_index("c")
    s = lax.axis_index("s")
    wid = c * NS + s
    base = s * RPS
    pltpu.sync_copy(zeros.at[pl.ds(base, RPS)], acc.at[pl.ds(base, RPS)])
    pltpu.sync_copy(dst3.at[wid], dstv)
    pltpu.sync_copy(ones128, onesv)
    plsc.subcore_barrier()

    @pl.loop(0, CH)
    def _(j):
        pltpu.sync_copy(onesv, acc.at[dstv.at[j]], add=True)

    plsc.subcore_barrier()
    pltpu.sync_copy(acc.at[pl.ds(base, RPS)], out.at[c, pl.ds(base, RPS)])


_deg = pl.kernel(
    _deg_body,
    out_type=jax.ShapeDtypeStruct((NC, NP, H), jnp.float32),
    mesh=plsc.VectorSubcoreMesh(core_axis_name="c", subcore_axis_name="s"),
    scratch_types=[
        pltpu.VMEM((CH, K), jnp.int32),
        pltpu.VMEM((K, H), jnp.float32),
        pltpu.VMEM_SHARED((NP, H), jnp.float32),
    ],
)


def _final_gather_body(h2, env2, bn, gf, envf, idxv, rows):
    c = lax.axis_index("c")
    s = lax.axis_index("s")
    wid = c * NS + s
    base = wid * BPW
    pltpu.sync_copy(bn.at[pl.ds(base, BPW)], idxv)
    pltpu.sync_copy(h2.at[idxv], rows)
    pltpu.sync_copy(rows, gf.at[pl.ds(base, BPW)])
    pltpu.sync_copy(env2.at[idxv], rows)
    pltpu.sync_copy(rows, envf.at[pl.ds(base, BPW)])


_final_gather = pl.kernel(
    _final_gather_body,
    out_type=(
        jax.ShapeDtypeStruct((BATCH, H), jnp.float32),
        jax.ShapeDtypeStruct((BATCH, H), jnp.float32),
    ),
    mesh=plsc.VectorSubcoreMesh(core_axis_name="c", subcore_axis_name="s"),
    scratch_types=[
        pltpu.VMEM((BPW,), jnp.int32),
        pltpu.VMEM((BPW, H), jnp.float32),
    ],
)


def _pre_body(x_ref, bn_ref, d0_ref, d1_ref, w1_ref, hwp_ref, hwpm_ref, dinv_ref):
    i = pl.program_id(0)
    deg = d0_ref[...] + d1_ref[...]
    dinv = lax.rsqrt(deg)
    ids = lax.broadcasted_iota(jnp.int32, (RB, 1), 0) + i * RB
    bn = bn_ref[...]
    hits = jnp.zeros((RB, 1), jnp.float32)
    for r in range(8):
        eq = (ids == bn[r:r + 1, :]).astype(jnp.float32)
        hits = hits + jnp.sum(eq, axis=1, keepdims=True)
    m = (hits == 0.0).astype(jnp.float32)
    xb = x_ref[...]
    w1 = w1_ref[...]
    hwp_ref[...] = jnp.dot(xb, w1, **_MM) * dinv
    hwpm_ref[...] = jnp.dot(xb * m, w1, **_MM) * dinv
    dinv_ref[...] = dinv


_pre = pl.pallas_call(
    _pre_body,
    grid=(NB,),
    in_specs=[
        pl.BlockSpec((RB, H), lambda i: (i, 0)),
        pl.BlockSpec((8, 128), lambda i: (0, 0)),
        pl.BlockSpec((RB, 1), lambda i: (i, 0)),
        pl.BlockSpec((RB, 1), lambda i: (i, 0)),
        pl.BlockSpec((H, H), lambda i: (0, 0)),
    ],
    out_specs=[
        pl.BlockSpec((RB, H), lambda i: (i, 0)),
        pl.BlockSpec((RB, H), lambda i: (i, 0)),
        pl.BlockSpec((RB, 1), lambda i: (i, 0)),
    ],
    out_shape=[
        jax.ShapeDtypeStruct((NP, H), jnp.float32),
        jax.ShapeDtypeStruct((NP, H), jnp.float32),
        jax.ShapeDtypeStruct((NP, 1), jnp.float32),
    ],
)


def _mid_body(p0_ref, p1_ref, hwp_ref, dinv_ref, b_ref, w_ref, out_ref):
    d = dinv_ref[...]
    h = jnp.maximum(d * (p0_ref[...] + p1_ref[...] - hwp_ref[...]) + b_ref[...], 0.0)
    out_ref[...] = jnp.dot(h, w_ref[...], **_MM) * d


_mid = pl.pallas_call(
    _mid_body,
    grid=(NB,),
    in_specs=[
        pl.BlockSpec((RB, H), lambda i: (i, 0)),
        pl.BlockSpec((RB, H), lambda i: (i, 0)),
        pl.BlockSpec((RB, H), lambda i: (i, 0)),
        pl.BlockSpec((RB, 1), lambda i: (i, 0)),
        pl.BlockSpec((1, H), lambda i: (0, 0)),
        pl.BlockSpec((H, H), lambda i: (0, 0)),
    ],
    out_specs=pl.BlockSpec((RB, H), lambda i: (i, 0)),
    out_shape=jax.ShapeDtypeStruct((NP, H), jnp.float32),
)


def _post_body(p0_ref, p1_ref, hwp_ref, dinv_ref, b_ref, out_ref):
    d = dinv_ref[...]
    out_ref[...] = jnp.maximum(
        d * (p0_ref[...] + p1_ref[...] - hwp_ref[...]) + b_ref[...], 0.0)


_post = pl.pallas_call(
    _post_body,
    grid=(NB,),
    in_specs=[
        pl.BlockSpec((RB, H), lambda i: (i, 0)),
        pl.BlockSpec((RB, H), lambda i: (i, 0)),
        pl.BlockSpec((RB, H), lambda i: (i, 0)),
        pl.BlockSpec((RB, 1), lambda i: (i, 0)),
        pl.BlockSpec((1, H), lambda i: (0, 0)),
    ],
    out_specs=pl.BlockSpec((RB, H), lambda i: (i, 0)),
    out_shape=jax.ShapeDtypeStruct((NP, H), jnp.float32),
)


def _bce(z, y):
    return jnp.mean(jnp.maximum(z, 0.0) - z * y + jnp.log1p(jnp.exp(-jnp.abs(z))))


def _loss_body(gf_ref, envf_ref, bn_ref, lab_ref, wc_ref, bc_ref, wl_ref, bl_ref,
               out_ref):
    g = gf_ref[...]
    e = envf_ref[...]
    bn = bn_ref[...]

    def lab_step(r, acc):
        ids = r * 128 + lax.broadcasted_iota(jnp.int32, (1, 128), 1)
        eq = (bn == ids).astype(jnp.float32)
        lr = lab_ref[pl.ds(r, 1), :]
        return acc + jnp.sum(eq * lr, axis=1, keepdims=True)

    y = lax.fori_loop(0, NP // 128, lab_step,
                      jnp.zeros((BATCH, 1), jnp.float32))
    spos = jnp.sum(y)
    proto = jnp.sum(g * y, axis=0, keepdims=True) / jnp.maximum(spos, 1.0)
    t = jnp.dot(g, wc_ref[...], **_MM)
    z1 = jnp.sum(t * proto, axis=1, keepdims=True) + bc_ref[0, 0]
    l1 = _bce(z1, y)
    u = jnp.dot(e, wl_ref[...], **_MM)
    z2 = jnp.sum(u * g, axis=1, keepdims=True) + bl_ref[0, 0]
    l2 = _bce(z2, 1.0 - y)
    out_ref[...] = jnp.reshape(0.5 * l1 + 0.5 * l2, (1, 1))


_loss = pl.pallas_call(
    _loss_body,
    out_shape=jax.ShapeDtypeStruct((1, 1), jnp.float32),
)


@jax.jit
def kernel(x, edge_index, batch_nodes, labels, W1, b1, W2, b2, Wc, bc, Wl, bl):
    src, dst = edge_index[0], edge_index[1]
    pad = EP - EDGES
    srcp = jnp.concatenate([src, jnp.zeros((pad,), jnp.int32)])
    dstp = jnp.concatenate([dst, jnp.full((pad,), NNODES, jnp.int32)])
    src3 = srcp.reshape(NW, CH, K)
    dst3 = dstp.reshape(NW, CH, K)
    xp = jnp.pad(x, ((0, NP - NNODES), (0, 0)))
    labp = jnp.pad(labels.astype(jnp.float32), (0, NP - NNODES))
    bn2d = batch_nodes.reshape(8, 128)

    degp = _deg(jnp.zeros((NP, H), jnp.float32),
                jnp.ones((K, H), jnp.float32), dst3)
    d0 = degp[0, :, 0:1]
    d1 = degp[1, :, 0:1] + 1.0

    hwp1, hwp1m, dinv = _pre(xp, bn2d, d0, d1, W1)

    b1r = b1.reshape(1, H)
    b2r = b2.reshape(1, H)
    p = _segsum_feat(hwp1, src3, dst3)
    q = _segsum_feat(hwp1m, src3, dst3)
    hwp2 = _mid(p[0], p[1], hwp1, dinv, b1r, W2)
    hwp2m = _mid(q[0], q[1], hwp1m, dinv, b1r, W2)
    p2 = _segsum_feat(hwp2, src3, dst3)
    q2 = _segsum_feat(hwp2m, src3, dst3)
    h2 = _post(p2[0], p2[1], hwp2, dinv, b2r)
    env2 = _post(q2[0], q2[1], hwp2m, dinv, b2r)

    gf, envf = _final_gather(h2, env2, batch_nodes)
    loss = _loss(gf, envf, batch_nodes.reshape(BATCH, 1), labp.reshape(NP // 128, 128),
                 Wc[0], bc.reshape(1, 1), Wl[0], bl.reshape(1, 1))
    return gf, loss[0, 0]
